# baseline (speedup 1.0000x reference)
import functools

import jax
import jax.numpy as jnp
from jax import lax
from jax.experimental import pallas as pl
from jax.experimental.pallas import tpu as pltpu
from jax.experimental.pallas import tpu_sc as plsc

_N = 10000
_E = 320000
_D = 128

_NC = 2
_NS = 16
_NW = _NC * _NS
_CHUNK = 128
_NR = 3
_NS_RING = 4
_ND_RING = 3
_UNROLL = 12
_EPT = _E // _NW
_CHUNKS = 84
_IDX_ROWS = 88
_EPT_PAD = _IDX_ROWS * _CHUNK
_ACC_ROWS = 10112
_ZROWS = _ACC_ROWS // _NS

_f32 = jnp.float32


def _sc_body(hw, src, dst, zeros, agg_out, src_r, dst_r, rows_v, acc_sh,
             g0, g1, g2, s0, s1, s2, s3, d0, d1, d2):
    gsems = (g0, g1, g2)
    ssems = (s0, s1, s2, s3)
    dsems = (d0, d1, d2)
    cid = lax.axis_index("c")
    sid = lax.axis_index("s")
    wid = sid * _NC + cid

    pltpu.sync_copy(zeros, acc_sh.at[pl.ds(sid * _ZROWS, _ZROWS)])
    plsc.subcore_barrier()

    for r in range(_NS_RING):
        pltpu.async_copy(src.at[wid, r], src_r.at[r], ssems[r])
    for r in range(_ND_RING):
        pltpu.async_copy(dst.at[wid, r], dst_r.at[r], dsems[r])
    for r in range(_NR):
        pltpu.make_async_copy(src.at[wid, r], src_r.at[r], ssems[r]).wait()
        pltpu.async_copy(hw.at[src_r.at[r]], rows_v.at[r], gsems[r])

    def group(g, carry):
        for u in range(_UNROLL):
            j = g * _UNROLL + u
            rs = u % _NR
            ds = u % _ND_RING
            ss = (u + 3) % _NS_RING
            pltpu.make_async_copy(dst.at[wid, j], dst_r.at[ds],
                                  dsems[ds]).wait()
            pltpu.make_async_copy(hw.at[src_r.at[u % _NS_RING]],
                                  rows_v.at[rs], gsems[rs]).wait()
            pltpu.sync_copy(rows_v.at[rs], acc_sh.at[dst_r.at[ds]],
                            add=True)
            pltpu.make_async_copy(src.at[wid, j + 3], src_r.at[ss],
                                  ssems[ss]).wait()
            pltpu.async_copy(hw.at[src_r.at[ss]], rows_v.at[rs], gsems[rs])
            pltpu.async_copy(src.at[wid, j + 4], src_r.at[u % _NS_RING],
                             ssems[u % _NS_RING])
            pltpu.async_copy(dst.at[wid, j + 3], dst_r.at[ds], dsems[ds])
        return carry

    lax.fori_loop(0, _CHUNKS // _UNROLL, group, 0)

    for c in range(_CHUNKS, _CHUNKS + _NR):
        pltpu.make_async_copy(hw.at[src_r.at[c % _NS_RING]],
                              rows_v.at[c % _NR], gsems[c % _NR]).wait()
    pltpu.make_async_copy(src.at[wid, _CHUNKS + 3],
                          src_r.at[(_CHUNKS + 3) % _NS_RING],
                          ssems[(_CHUNKS + 3) % _NS_RING]).wait()
    for k in range(_ND_RING):
        pltpu.make_async_copy(dst.at[wid, _CHUNKS + k], dst_r.at[k],
                              dsems[k]).wait()
    plsc.subcore_barrier()

    pltpu.sync_copy(acc_sh.at[pl.ds(sid * _ZROWS, _ZROWS)],
                    agg_out.at[cid, pl.ds(sid * _ZROWS, _ZROWS)])


def _make_sc():
    mesh = plsc.VectorSubcoreMesh(core_axis_name="c", subcore_axis_name="s")
    out_type = jax.ShapeDtypeStruct((_NC, _ACC_ROWS, _D), _f32)
    scratch = [
        pltpu.VMEM((_NS_RING, _CHUNK), jnp.int32),
        pltpu.VMEM((_ND_RING, _CHUNK), jnp.int32),
        pltpu.VMEM((_NR, _CHUNK, _D), _f32),
        pltpu.VMEM_SHARED((_ACC_ROWS, _D), _f32),
    ] + [pltpu.SemaphoreType.DMA] * 10
    return pl.kernel(_sc_body, out_type=out_type, mesh=mesh,
                     scratch_types=scratch)


def _deg_body(dst, zeros, ones, deg_out, dst_v, ones_v, deg_sh):
    cid = lax.axis_index("c")
    sid = lax.axis_index("s")
    wid = sid * _NC + cid

    pltpu.sync_copy(zeros, deg_sh.at[pl.ds(sid * _ZROWS, _ZROWS)])
    pltpu.sync_copy(dst.at[wid], dst_v)
    pltpu.sync_copy(ones, ones_v)
    plsc.subcore_barrier()

    def step(j, carry):
        pltpu.sync_copy(ones_v, deg_sh.at[dst_v.at[j]], add=True)
        return carry

    lax.fori_loop(0, _CHUNKS, step, 0)
    plsc.subcore_barrier()

    pltpu.sync_copy(deg_sh.at[pl.ds(sid * _ZROWS, _ZROWS)],
                    deg_out.at[cid, pl.ds(sid * _ZROWS, _ZROWS)])


def _make_deg():
    mesh = plsc.VectorSubcoreMesh(core_axis_name="c", subcore_axis_name="s")
    out_type = jax.ShapeDtypeStruct((_NC, _ACC_ROWS, _D), _f32)
    scratch = [
        pltpu.VMEM((_IDX_ROWS, _CHUNK), jnp.int32),
        pltpu.VMEM((_CHUNK, _D), _f32),
        pltpu.VMEM_SHARED((_ACC_ROWS, _D), _f32),
    ]
    return pl.kernel(_deg_body, out_type=out_type, mesh=mesh,
                     scratch_types=scratch)


_BR = 2000


def _mm2_body(x_ref, wa_ref, wb_ref, a_ref, b_ref):
    x = x_ref[...]
    dn = (((1,), (1,)), ((), ()))
    a_ref[...] = lax.dot_general(x, wa_ref[...], dn,
                                 preferred_element_type=_f32)
    b_ref[...] = lax.dot_general(x, wb_ref[...], dn,
                                 preferred_element_type=_f32)


def _mm2(x, wa, wb):
    grid = (_N // _BR,)
    blk_x = pl.BlockSpec((_BR, _D), lambda i: (i, 0))
    blk_w = pl.BlockSpec((_D, _D), lambda i: (0, 0))
    return pl.pallas_call(
        _mm2_body,
        grid=grid,
        in_specs=[blk_x, blk_w, blk_w],
        out_specs=[blk_x, blk_x],
        out_shape=[jax.ShapeDtypeStruct((_N, _D), _f32)] * 2,
    )(x, wa, wb)


def _mid_body(xs_ref, agg_ref, deg_ref, b_ref, wa_ref, wb_ref,
              a_ref, b_out_ref):
    deg = deg_ref[0, :, 0:1] + deg_ref[1, :, 0:1]
    recip = 1.0 / jnp.maximum(deg, 1.0)
    h = xs_ref[...] + (agg_ref[0] + agg_ref[1]) * recip + b_ref[...]
    h = jnp.maximum(h, 0.0)
    dn = (((1,), (1,)), ((), ()))
    a_ref[...] = lax.dot_general(h, wa_ref[...], dn,
                                 preferred_element_type=_f32)
    b_out_ref[...] = lax.dot_general(h, wb_ref[...], dn,
                                     preferred_element_type=_f32)


def _mid(xs, agg, deg, b, wa, wb):
    grid = (_N // _BR,)
    blk_r = pl.BlockSpec((_BR, _D), lambda i: (i, 0))
    blk_a = pl.BlockSpec((_NC, _BR, _D), lambda i: (0, i, 0))
    blk_d = pl.BlockSpec((_NC, _BR, 16), lambda i: (0, i, 0))
    blk_b = pl.BlockSpec((1, _D), lambda i: (0, 0))
    blk_w = pl.BlockSpec((_D, _D), lambda i: (0, 0))
    return pl.pallas_call(
        _mid_body,
        grid=grid,
        in_specs=[blk_r, blk_a, blk_d, blk_b, blk_w, blk_w],
        out_specs=[blk_r, blk_r],
        out_shape=[jax.ShapeDtypeStruct((_N, _D), _f32)] * 2,
    )(xs, agg, deg, b, wa, wb)


def _fin_body(xs_ref, agg_ref, deg_ref, b_ref, o_ref):
    deg = deg_ref[0, :, 0:1] + deg_ref[1, :, 0:1]
    recip = 1.0 / jnp.maximum(deg, 1.0)
    o_ref[...] = xs_ref[...] + (agg_ref[0] + agg_ref[1]) * recip + b_ref[...]


def _fin(xs, agg, deg, b):
    grid = (_N // _BR,)
    blk_r = pl.BlockSpec((_BR, _D), lambda i: (i, 0))
    blk_a = pl.BlockSpec((_NC, _BR, _D), lambda i: (0, i, 0))
    blk_d = pl.BlockSpec((_NC, _BR, 16), lambda i: (0, i, 0))
    blk_b = pl.BlockSpec((1, _D), lambda i: (0, 0))
    return pl.pallas_call(
        _fin_body,
        grid=grid,
        in_specs=[blk_r, blk_a, blk_d, blk_b],
        out_specs=blk_r,
        out_shape=jax.ShapeDtypeStruct((_N, _D), _f32),
    )(xs, agg, deg, b)


def kernel(x, edge_index, W_self1, W_neigh1, b1, W_self2, W_neigh2, b2):
    src = edge_index[0].reshape(_NW, _EPT)
    dst = edge_index[1].reshape(_NW, _EPT)
    pad = _EPT_PAD - _EPT
    src_p = jnp.concatenate(
        [src, jnp.zeros((_NW, pad), jnp.int32)], axis=1
    ).reshape(_NW, _IDX_ROWS, _CHUNK)
    dst_p = jnp.concatenate(
        [dst, jnp.full((_NW, pad), _N, jnp.int32)], axis=1
    ).reshape(_NW, _IDX_ROWS, _CHUNK)

    zeros = jnp.zeros((_ZROWS, _D), _f32)
    ones = jnp.ones((_CHUNK, _D), _f32)

    xs1, hw1 = _mm2(x, W_self1, W_neigh1)
    deg = _make_deg()(dst_p, zeros, ones)[:, :, :16]
    agg1 = _make_sc()(hw1, src_p, dst_p, zeros)
    xs2, hw2 = _mid(xs1, agg1, deg, b1.reshape(1, _D), W_self2, W_neigh2)
    agg2 = _make_sc()(hw2, src_p, dst_p, zeros)
    return _fin(xs2, agg2, deg, b2.reshape(1, _D))


# speedup vs baseline: 1.9898x; 1.9898x over previous
import functools

import jax
import jax.numpy as jnp
from jax import lax
from jax.experimental import pallas as pl
from jax.experimental.pallas import tpu as pltpu
from jax.experimental.pallas import tpu_sc as plsc

_N = 10000
_E = 320000
_D = 128

_NC = 2
_NS = 16
_NW = _NC * _NS
_CHUNK = 128
_NSUB = 2
_EPT = _E // _NW
_CHUNKS = 80
_IDX_ROWS = 88
_EPT_PAD = _IDX_ROWS * _CHUNK
_ACC_ROWS = 10112
_ZROWS = _ACC_ROWS // _NS

_f32 = jnp.float32


def _decode(idx_v, src_r, dst_r, row, slot):
    for k in range(_CHUNK // 16):
        w = idx_v[row, pl.ds(16 * k, 16)]
        src_r[slot, pl.ds(16 * k, 16)] = lax.bitwise_and(w, 0x3FFF)
        dst_r[slot, pl.ds(16 * k, 16)] = lax.shift_right_logical(w, 14)


def _gather(hw, src_r, rows_v, slot, sem):
    q = _CHUNK // _NSUB
    for g in range(_NSUB):
        pltpu.async_copy(hw.at[src_r.at[slot, pl.ds(g * q, q)]],
                         rows_v.at[slot, pl.ds(g * q, q)], sem)


def _sc_body(hw, packed, zeros, agg_out, idx_v, src_r, dst_r, rows_v,
             acc_sh, g0, g1):
    gsems = (g0, g1)
    cid = lax.axis_index("c")
    sid = lax.axis_index("s")
    wid = sid * _NC + cid

    pltpu.sync_copy(zeros, acc_sh.at[pl.ds(sid * _ZROWS, _ZROWS)])
    pltpu.sync_copy(packed.at[wid], idx_v)
    plsc.subcore_barrier()

    for p in range(2):
        _decode(idx_v, src_r, dst_r, p, p)
        _gather(hw, src_r, rows_v, p, gsems[p])

    def group(g, carry):
        for u in range(2):
            j = g * 2 + u
            pltpu.make_async_copy(hw.at[src_r.at[u]], rows_v.at[u],
                                  gsems[u]).wait()
            pltpu.sync_copy(rows_v.at[u], acc_sh.at[dst_r.at[u]],
                            add=True)
            _decode(idx_v, src_r, dst_r, j + 2, u)
            _gather(hw, src_r, rows_v, u, gsems[u])
        return carry

    lax.fori_loop(0, _CHUNKS // 2, group, 0)

    for u in range(2):
        pltpu.make_async_copy(hw.at[src_r.at[u]], rows_v.at[u],
                              gsems[u]).wait()
    plsc.subcore_barrier()

    pltpu.sync_copy(acc_sh.at[pl.ds(sid * _ZROWS, _ZROWS)],
                    agg_out.at[cid, pl.ds(sid * _ZROWS, _ZROWS)])


def _make_sc():
    mesh = plsc.VectorSubcoreMesh(core_axis_name="c", subcore_axis_name="s")
    out_type = jax.ShapeDtypeStruct((_NC, _ACC_ROWS, _D), _f32)
    scratch = [
        pltpu.VMEM((_IDX_ROWS, _CHUNK), jnp.int32),
        pltpu.VMEM((2, _CHUNK), jnp.int32),
        pltpu.VMEM((2, _CHUNK), jnp.int32),
        pltpu.VMEM((2, _CHUNK, _D), _f32),
        pltpu.VMEM_SHARED((_ACC_ROWS, _D), _f32),
        pltpu.SemaphoreType.DMA,
        pltpu.SemaphoreType.DMA,
    ]
    return pl.kernel(_sc_body, out_type=out_type, mesh=mesh,
                     scratch_types=scratch)


def _deg_body(dst, zeros, ones, deg_out, dst_v, ones_v, deg_sh):
    cid = lax.axis_index("c")
    sid = lax.axis_index("s")
    wid = sid * _NC + cid

    pltpu.sync_copy(zeros, deg_sh.at[pl.ds(sid * _ZROWS, _ZROWS)])
    pltpu.sync_copy(dst.at[wid], dst_v)
    pltpu.sync_copy(ones, ones_v)
    plsc.subcore_barrier()

    def step(j, carry):
        pltpu.sync_copy(ones_v, deg_sh.at[dst_v.at[j]], add=True)
        return carry

    lax.fori_loop(0, _CHUNKS, step, 0)
    plsc.subcore_barrier()

    pltpu.sync_copy(deg_sh.at[pl.ds(sid * _ZROWS, _ZROWS)],
                    deg_out.at[cid, pl.ds(sid * _ZROWS, _ZROWS)])


def _make_deg():
    mesh = plsc.VectorSubcoreMesh(core_axis_name="c", subcore_axis_name="s")
    out_type = jax.ShapeDtypeStruct((_NC, _ACC_ROWS, _D), _f32)
    scratch = [
        pltpu.VMEM((_IDX_ROWS, _CHUNK), jnp.int32),
        pltpu.VMEM((_CHUNK, _D), _f32),
        pltpu.VMEM_SHARED((_ACC_ROWS, _D), _f32),
    ]
    return pl.kernel(_deg_body, out_type=out_type, mesh=mesh,
                     scratch_types=scratch)


_BR = 2000


def _mm2_body(x_ref, wa_ref, wb_ref, a_ref, b_ref):
    x = x_ref[...]
    dn = (((1,), (1,)), ((), ()))
    a_ref[...] = lax.dot_general(x, wa_ref[...], dn,
                                 preferred_element_type=_f32)
    b_ref[...] = lax.dot_general(x, wb_ref[...], dn,
                                 preferred_element_type=_f32)


def _mm2(x, wa, wb):
    grid = (_N // _BR,)
    blk_x = pl.BlockSpec((_BR, _D), lambda i: (i, 0))
    blk_w = pl.BlockSpec((_D, _D), lambda i: (0, 0))
    return pl.pallas_call(
        _mm2_body,
        grid=grid,
        in_specs=[blk_x, blk_w, blk_w],
        out_specs=[blk_x, blk_x],
        out_shape=[jax.ShapeDtypeStruct((_N, _D), _f32)] * 2,
    )(x, wa, wb)


def _mid_body(xs_ref, agg_ref, deg_ref, b_ref, wa_ref, wb_ref,
              a_ref, b_out_ref):
    deg = deg_ref[0, :, 0:1] + deg_ref[1, :, 0:1]
    recip = 1.0 / jnp.maximum(deg, 1.0)
    h = xs_ref[...] + (agg_ref[0] + agg_ref[1]) * recip + b_ref[...]
    h = jnp.maximum(h, 0.0)
    dn = (((1,), (1,)), ((), ()))
    a_ref[...] = lax.dot_general(h, wa_ref[...], dn,
                                 preferred_element_type=_f32)
    b_out_ref[...] = lax.dot_general(h, wb_ref[...], dn,
                                     preferred_element_type=_f32)


def _mid(xs, agg, deg, b, wa, wb):
    grid = (_N // _BR,)
    blk_r = pl.BlockSpec((_BR, _D), lambda i: (i, 0))
    blk_a = pl.BlockSpec((_NC, _BR, _D), lambda i: (0, i, 0))
    blk_d = pl.BlockSpec((_NC, _BR, 16), lambda i: (0, i, 0))
    blk_b = pl.BlockSpec((1, _D), lambda i: (0, 0))
    blk_w = pl.BlockSpec((_D, _D), lambda i: (0, 0))
    return pl.pallas_call(
        _mid_body,
        grid=grid,
        in_specs=[blk_r, blk_a, blk_d, blk_b, blk_w, blk_w],
        out_specs=[blk_r, blk_r],
        out_shape=[jax.ShapeDtypeStruct((_N, _D), _f32)] * 2,
    )(xs, agg, deg, b, wa, wb)


def _fin_body(xs_ref, agg_ref, deg_ref, b_ref, o_ref):
    deg = deg_ref[0, :, 0:1] + deg_ref[1, :, 0:1]
    recip = 1.0 / jnp.maximum(deg, 1.0)
    o_ref[...] = xs_ref[...] + (agg_ref[0] + agg_ref[1]) * recip + b_ref[...]


def _fin(xs, agg, deg, b):
    grid = (_N // _BR,)
    blk_r = pl.BlockSpec((_BR, _D), lambda i: (i, 0))
    blk_a = pl.BlockSpec((_NC, _BR, _D), lambda i: (0, i, 0))
    blk_d = pl.BlockSpec((_NC, _BR, 16), lambda i: (0, i, 0))
    blk_b = pl.BlockSpec((1, _D), lambda i: (0, 0))
    return pl.pallas_call(
        _fin_body,
        grid=grid,
        in_specs=[blk_r, blk_a, blk_d, blk_b],
        out_specs=blk_r,
        out_shape=jax.ShapeDtypeStruct((_N, _D), _f32),
    )(xs, agg, deg, b)


def kernel(x, edge_index, W_self1, W_neigh1, b1, W_self2, W_neigh2, b2):
    src = edge_index[0].reshape(_NW, _EPT)
    dst = edge_index[1].reshape(_NW, _EPT)
    pad = _EPT_PAD - _EPT
    src_p = jnp.concatenate(
        [src, jnp.zeros((_NW, pad), jnp.int32)], axis=1
    ).reshape(_NW, _IDX_ROWS, _CHUNK)
    dst_p = jnp.concatenate(
        [dst, jnp.full((_NW, pad), _N, jnp.int32)], axis=1
    ).reshape(_NW, _IDX_ROWS, _CHUNK)
    packed_p = jnp.bitwise_or(src_p, jnp.left_shift(dst_p, 14))

    zeros = jnp.zeros((_ZROWS, _D), _f32)
    ones = jnp.ones((_CHUNK, _D), _f32)

    xs1, hw1 = _mm2(x, W_self1, W_neigh1)
    deg = _make_deg()(dst_p, zeros, ones)[:, :, :16]
    agg1 = _make_sc()(hw1, packed_p, zeros)
    xs2, hw2 = _mid(xs1, agg1, deg, b1.reshape(1, _D), W_self2, W_neigh2)
    agg2 = _make_sc()(hw2, packed_p, zeros)
    return _fin(xs2, agg2, deg, b2.reshape(1, _D))


# speedup vs baseline: 1.9910x; 1.0006x over previous
import functools

import jax
import jax.numpy as jnp
from jax import lax
from jax.experimental import pallas as pl
from jax.experimental.pallas import tpu as pltpu
from jax.experimental.pallas import tpu_sc as plsc

_N = 10000
_E = 320000
_D = 128

_NC = 2
_NS = 16
_NW = _NC * _NS
_CHUNK = 128
_NSUB = 1
_EPT = _E // _NW
_CHUNKS = 80
_IDX_ROWS = 88
_EPT_PAD = _IDX_ROWS * _CHUNK
_ACC_ROWS = 10112
_ZROWS = _ACC_ROWS // _NS

_f32 = jnp.float32


def _decode(idx_v, src_r, dst_r, row, slot):
    for k in range(_CHUNK // 16):
        w = idx_v[row, pl.ds(16 * k, 16)]
        src_r[slot, pl.ds(16 * k, 16)] = lax.bitwise_and(w, 0x3FFF)
        dst_r[slot, pl.ds(16 * k, 16)] = lax.shift_right_logical(w, 14)


def _gather(hw, src_r, rows_v, slot, sem):
    q = _CHUNK // _NSUB
    for g in range(_NSUB):
        pltpu.async_copy(hw.at[src_r.at[slot, pl.ds(g * q, q)]],
                         rows_v.at[slot, pl.ds(g * q, q)], sem)


def _sc_body(hw, packed, zeros, agg_out, idx_v, src_r, dst_r, rows_v,
             acc_sh, g0, g1):
    gsems = (g0, g1)
    cid = lax.axis_index("c")
    sid = lax.axis_index("s")
    wid = sid * _NC + cid

    pltpu.sync_copy(zeros, acc_sh.at[pl.ds(sid * _ZROWS, _ZROWS)])
    pltpu.sync_copy(packed.at[wid], idx_v)
    plsc.subcore_barrier()

    for p in range(2):
        _decode(idx_v, src_r, dst_r, p, p)
        _gather(hw, src_r, rows_v, p, gsems[p])

    def group(g, carry):
        for u in range(2):
            j = g * 2 + u
            pltpu.make_async_copy(hw.at[src_r.at[u]], rows_v.at[u],
                                  gsems[u]).wait()
            pltpu.sync_copy(rows_v.at[u], acc_sh.at[dst_r.at[u]],
                            add=True)
            _decode(idx_v, src_r, dst_r, j + 2, u)
            _gather(hw, src_r, rows_v, u, gsems[u])
        return carry

    lax.fori_loop(0, _CHUNKS // 2, group, 0)

    for u in range(2):
        pltpu.make_async_copy(hw.at[src_r.at[u]], rows_v.at[u],
                              gsems[u]).wait()
    plsc.subcore_barrier()

    pltpu.sync_copy(acc_sh.at[pl.ds(sid * _ZROWS, _ZROWS)],
                    agg_out.at[cid, pl.ds(sid * _ZROWS, _ZROWS)])


def _make_sc():
    mesh = plsc.VectorSubcoreMesh(core_axis_name="c", subcore_axis_name="s")
    out_type = jax.ShapeDtypeStruct((_NC, _ACC_ROWS, _D), _f32)
    scratch = [
        pltpu.VMEM((_IDX_ROWS, _CHUNK), jnp.int32),
        pltpu.VMEM((2, _CHUNK), jnp.int32),
        pltpu.VMEM((2, _CHUNK), jnp.int32),
        pltpu.VMEM((2, _CHUNK, _D), _f32),
        pltpu.VMEM_SHARED((_ACC_ROWS, _D), _f32),
        pltpu.SemaphoreType.DMA,
        pltpu.SemaphoreType.DMA,
    ]
    return pl.kernel(_sc_body, out_type=out_type, mesh=mesh,
                     scratch_types=scratch)


def _deg_body(dst, zeros, ones, deg_out, dst_v, ones_v, deg_sh):
    cid = lax.axis_index("c")
    sid = lax.axis_index("s")
    wid = sid * _NC + cid

    pltpu.sync_copy(zeros, deg_sh.at[pl.ds(sid * _ZROWS, _ZROWS)])
    pltpu.sync_copy(dst.at[wid], dst_v)
    pltpu.sync_copy(ones, ones_v)
    plsc.subcore_barrier()

    def step(j, carry):
        pltpu.sync_copy(ones_v, deg_sh.at[dst_v.at[j]], add=True)
        return carry

    lax.fori_loop(0, _CHUNKS, step, 0)
    plsc.subcore_barrier()

    pltpu.sync_copy(deg_sh.at[pl.ds(sid * _ZROWS, _ZROWS)],
                    deg_out.at[cid, pl.ds(sid * _ZROWS, _ZROWS)])


def _make_deg():
    mesh = plsc.VectorSubcoreMesh(core_axis_name="c", subcore_axis_name="s")
    out_type = jax.ShapeDtypeStruct((_NC, _ACC_ROWS, _D), _f32)
    scratch = [
        pltpu.VMEM((_IDX_ROWS, _CHUNK), jnp.int32),
        pltpu.VMEM((_CHUNK, _D), _f32),
        pltpu.VMEM_SHARED((_ACC_ROWS, _D), _f32),
    ]
    return pl.kernel(_deg_body, out_type=out_type, mesh=mesh,
                     scratch_types=scratch)


_BR = 2000


def _mm2_body(x_ref, wa_ref, wb_ref, a_ref, b_ref):
    x = x_ref[...]
    dn = (((1,), (1,)), ((), ()))
    a_ref[...] = lax.dot_general(x, wa_ref[...], dn,
                                 preferred_element_type=_f32)
    b_ref[...] = lax.dot_general(x, wb_ref[...], dn,
                                 preferred_element_type=_f32)


def _mm2(x, wa, wb):
    grid = (_N // _BR,)
    blk_x = pl.BlockSpec((_BR, _D), lambda i: (i, 0))
    blk_w = pl.BlockSpec((_D, _D), lambda i: (0, 0))
    return pl.pallas_call(
        _mm2_body,
        grid=grid,
        in_specs=[blk_x, blk_w, blk_w],
        out_specs=[blk_x, blk_x],
        out_shape=[jax.ShapeDtypeStruct((_N, _D), _f32)] * 2,
    )(x, wa, wb)


def _mid_body(xs_ref, agg_ref, deg_ref, b_ref, wa_ref, wb_ref,
              a_ref, b_out_ref):
    deg = deg_ref[0, :, 0:1] + deg_ref[1, :, 0:1]
    recip = 1.0 / jnp.maximum(deg, 1.0)
    h = xs_ref[...] + (agg_ref[0] + agg_ref[1]) * recip + b_ref[...]
    h = jnp.maximum(h, 0.0)
    dn = (((1,), (1,)), ((), ()))
    a_ref[...] = lax.dot_general(h, wa_ref[...], dn,
                                 preferred_element_type=_f32)
    b_out_ref[...] = lax.dot_general(h, wb_ref[...], dn,
                                     preferred_element_type=_f32)


def _mid(xs, agg, deg, b, wa, wb):
    grid = (_N // _BR,)
    blk_r = pl.BlockSpec((_BR, _D), lambda i: (i, 0))
    blk_a = pl.BlockSpec((_NC, _BR, _D), lambda i: (0, i, 0))
    blk_d = pl.BlockSpec((_NC, _BR, 16), lambda i: (0, i, 0))
    blk_b = pl.BlockSpec((1, _D), lambda i: (0, 0))
    blk_w = pl.BlockSpec((_D, _D), lambda i: (0, 0))
    return pl.pallas_call(
        _mid_body,
        grid=grid,
        in_specs=[blk_r, blk_a, blk_d, blk_b, blk_w, blk_w],
        out_specs=[blk_r, blk_r],
        out_shape=[jax.ShapeDtypeStruct((_N, _D), _f32)] * 2,
    )(xs, agg, deg, b, wa, wb)


def _fin_body(xs_ref, agg_ref, deg_ref, b_ref, o_ref):
    deg = deg_ref[0, :, 0:1] + deg_ref[1, :, 0:1]
    recip = 1.0 / jnp.maximum(deg, 1.0)
    o_ref[...] = xs_ref[...] + (agg_ref[0] + agg_ref[1]) * recip + b_ref[...]


def _fin(xs, agg, deg, b):
    grid = (_N // _BR,)
    blk_r = pl.BlockSpec((_BR, _D), lambda i: (i, 0))
    blk_a = pl.BlockSpec((_NC, _BR, _D), lambda i: (0, i, 0))
    blk_d = pl.BlockSpec((_NC, _BR, 16), lambda i: (0, i, 0))
    blk_b = pl.BlockSpec((1, _D), lambda i: (0, 0))
    return pl.pallas_call(
        _fin_body,
        grid=grid,
        in_specs=[blk_r, blk_a, blk_d, blk_b],
        out_specs=blk_r,
        out_shape=jax.ShapeDtypeStruct((_N, _D), _f32),
    )(xs, agg, deg, b)


def kernel(x, edge_index, W_self1, W_neigh1, b1, W_self2, W_neigh2, b2):
    src = edge_index[0].reshape(_NW, _EPT)
    dst = edge_index[1].reshape(_NW, _EPT)
    pad = _EPT_PAD - _EPT
    src_p = jnp.concatenate(
        [src, jnp.zeros((_NW, pad), jnp.int32)], axis=1
    ).reshape(_NW, _IDX_ROWS, _CHUNK)
    dst_p = jnp.concatenate(
        [dst, jnp.full((_NW, pad), _N, jnp.int32)], axis=1
    ).reshape(_NW, _IDX_ROWS, _CHUNK)
    packed_p = jnp.bitwise_or(src_p, jnp.left_shift(dst_p, 14))

    zeros = jnp.zeros((_ZROWS, _D), _f32)
    ones = jnp.ones((_CHUNK, _D), _f32)

    xs1, hw1 = _mm2(x, W_self1, W_neigh1)
    deg = _make_deg()(dst_p, zeros, ones)[:, :, :16]
    agg1 = _make_sc()(hw1, packed_p, zeros)
    xs2, hw2 = _mid(xs1, agg1, deg, b1.reshape(1, _D), W_self2, W_neigh2)
    agg2 = _make_sc()(hw2, packed_p, zeros)
    return _fin(xs2, agg2, deg, b2.reshape(1, _D))


# speedup vs baseline: 3.0278x; 1.5208x over previous
import functools

import jax
import jax.numpy as jnp
from jax import lax
from jax.experimental import pallas as pl
from jax.experimental.pallas import tpu as pltpu
from jax.experimental.pallas import tpu_sc as plsc

_N = 10000
_E = 320000
_D = 128

_NC = 2
_NS = 16
_NW = _NC * _NS
_CHUNK = 128
_EPT = _E // _NW
_CHUNKS = 80
_IDX_ROWS = 88
_EPT_PAD = _IDX_ROWS * _CHUNK
_ACC_ROWS = 10112
_ZROWS = _ACC_ROWS // _NS

_f32 = jnp.float32


def _decode(idx_v, src_r, dst_r, row, slot):
    for k in range(_CHUNK // 16):
        w = idx_v[row, pl.ds(16 * k, 16)]
        src_r[slot, pl.ds(16 * k, 16)] = lax.bitwise_and(w, 0x3FFF)
        dst_r[slot, pl.ds(16 * k, 16)] = lax.shift_right_logical(w, 14)


def _sc_body(hw, packed, zeros, agg_out, idx_v, src_r, dst_r, rows_v,
             acc_sh, g0, g1):
    gsems = (g0, g1)
    cid = lax.axis_index("c")
    sid = lax.axis_index("s")
    wid = sid * _NC + cid

    pltpu.sync_copy(zeros, acc_sh.at[pl.ds(sid * _ZROWS, _ZROWS)])
    pltpu.sync_copy(packed.at[wid], idx_v)
    plsc.subcore_barrier()

    for p in range(2):
        _decode(idx_v, src_r, dst_r, p, p)

    def group(g, carry):
        descs = [pltpu.async_copy(hw.at[src_r.at[u]], rows_v.at[u],
                                  gsems[u]) for u in range(2)]
        for u in range(2):
            descs[u].wait()
            pltpu.sync_copy(rows_v.at[u], acc_sh.at[dst_r.at[u]],
                            add=True)
            _decode(idx_v, src_r, dst_r, g * 2 + 2 + u, u)
        return carry

    lax.fori_loop(0, _CHUNKS // 2, group, 0)
    plsc.subcore_barrier()

    pltpu.sync_copy(acc_sh.at[pl.ds(sid * _ZROWS, _ZROWS)],
                    agg_out.at[cid, pl.ds(sid * _ZROWS, _ZROWS)])


def _make_sc():
    mesh = plsc.VectorSubcoreMesh(core_axis_name="c", subcore_axis_name="s")
    out_type = jax.ShapeDtypeStruct((_NC, _ACC_ROWS, _D), _f32)
    scratch = [
        pltpu.VMEM((_IDX_ROWS, _CHUNK), jnp.int32),
        pltpu.VMEM((2, _CHUNK), jnp.int32),
        pltpu.VMEM((2, _CHUNK), jnp.int32),
        pltpu.VMEM((2, _CHUNK, _D), _f32),
        pltpu.VMEM_SHARED((_ACC_ROWS, _D), _f32),
        pltpu.SemaphoreType.DMA,
        pltpu.SemaphoreType.DMA,
    ]
    return pl.kernel(_sc_body, out_type=out_type, mesh=mesh,
                     scratch_types=scratch)


def _deg_body(dst, zeros, ones, deg_out, dst_v, ones_v, deg_sh):
    cid = lax.axis_index("c")
    sid = lax.axis_index("s")
    wid = sid * _NC + cid

    pltpu.sync_copy(zeros, deg_sh.at[pl.ds(sid * _ZROWS, _ZROWS)])
    pltpu.sync_copy(dst.at[wid], dst_v)
    pltpu.sync_copy(ones, ones_v)
    plsc.subcore_barrier()

    def step(j, carry):
        pltpu.sync_copy(ones_v, deg_sh.at[dst_v.at[j]], add=True)
        return carry

    lax.fori_loop(0, _CHUNKS, step, 0)
    plsc.subcore_barrier()

    pltpu.sync_copy(deg_sh.at[pl.ds(sid * _ZROWS, _ZROWS)],
                    deg_out.at[cid, pl.ds(sid * _ZROWS, _ZROWS)])


def _make_deg():
    mesh = plsc.VectorSubcoreMesh(core_axis_name="c", subcore_axis_name="s")
    out_type = jax.ShapeDtypeStruct((_NC, _ACC_ROWS, _D), _f32)
    scratch = [
        pltpu.VMEM((_IDX_ROWS, _CHUNK), jnp.int32),
        pltpu.VMEM((_CHUNK, _D), _f32),
        pltpu.VMEM_SHARED((_ACC_ROWS, _D), _f32),
    ]
    return pl.kernel(_deg_body, out_type=out_type, mesh=mesh,
                     scratch_types=scratch)


_BR = 2000


def _mm2_body(x_ref, wa_ref, wb_ref, a_ref, b_ref):
    x = x_ref[...]
    dn = (((1,), (1,)), ((), ()))
    a_ref[...] = lax.dot_general(x, wa_ref[...], dn,
                                 preferred_element_type=_f32)
    b_ref[...] = lax.dot_general(x, wb_ref[...], dn,
                                 preferred_element_type=_f32)


def _mm2(x, wa, wb):
    grid = (_N // _BR,)
    blk_x = pl.BlockSpec((_BR, _D), lambda i: (i, 0))
    blk_w = pl.BlockSpec((_D, _D), lambda i: (0, 0))
    return pl.pallas_call(
        _mm2_body,
        grid=grid,
        in_specs=[blk_x, blk_w, blk_w],
        out_specs=[blk_x, blk_x],
        out_shape=[jax.ShapeDtypeStruct((_N, _D), _f32)] * 2,
    )(x, wa, wb)


def _mid_body(xs_ref, agg_ref, deg_ref, b_ref, wa_ref, wb_ref,
              a_ref, b_out_ref):
    deg = deg_ref[0, :, 0:1] + deg_ref[1, :, 0:1]
    recip = 1.0 / jnp.maximum(deg, 1.0)
    h = xs_ref[...] + (agg_ref[0] + agg_ref[1]) * recip + b_ref[...]
    h = jnp.maximum(h, 0.0)
    dn = (((1,), (1,)), ((), ()))
    a_ref[...] = lax.dot_general(h, wa_ref[...], dn,
                                 preferred_element_type=_f32)
    b_out_ref[...] = lax.dot_general(h, wb_ref[...], dn,
                                     preferred_element_type=_f32)


def _mid(xs, agg, deg, b, wa, wb):
    grid = (_N // _BR,)
    blk_r = pl.BlockSpec((_BR, _D), lambda i: (i, 0))
    blk_a = pl.BlockSpec((_NC, _BR, _D), lambda i: (0, i, 0))
    blk_d = pl.BlockSpec((_NC, _BR, 16), lambda i: (0, i, 0))
    blk_b = pl.BlockSpec((1, _D), lambda i: (0, 0))
    blk_w = pl.BlockSpec((_D, _D), lambda i: (0, 0))
    return pl.pallas_call(
        _mid_body,
        grid=grid,
        in_specs=[blk_r, blk_a, blk_d, blk_b, blk_w, blk_w],
        out_specs=[blk_r, blk_r],
        out_shape=[jax.ShapeDtypeStruct((_N, _D), _f32)] * 2,
    )(xs, agg, deg, b, wa, wb)


def _fin_body(xs_ref, agg_ref, deg_ref, b_ref, o_ref):
    deg = deg_ref[0, :, 0:1] + deg_ref[1, :, 0:1]
    recip = 1.0 / jnp.maximum(deg, 1.0)
    o_ref[...] = xs_ref[...] + (agg_ref[0] + agg_ref[1]) * recip + b_ref[...]


def _fin(xs, agg, deg, b):
    grid = (_N // _BR,)
    blk_r = pl.BlockSpec((_BR, _D), lambda i: (i, 0))
    blk_a = pl.BlockSpec((_NC, _BR, _D), lambda i: (0, i, 0))
    blk_d = pl.BlockSpec((_NC, _BR, 16), lambda i: (0, i, 0))
    blk_b = pl.BlockSpec((1, _D), lambda i: (0, 0))
    return pl.pallas_call(
        _fin_body,
        grid=grid,
        in_specs=[blk_r, blk_a, blk_d, blk_b],
        out_specs=blk_r,
        out_shape=jax.ShapeDtypeStruct((_N, _D), _f32),
    )(xs, agg, deg, b)


def kernel(x, edge_index, W_self1, W_neigh1, b1, W_self2, W_neigh2, b2):
    src = edge_index[0].reshape(_NW, _EPT)
    dst = edge_index[1].reshape(_NW, _EPT)
    pad = _EPT_PAD - _EPT
    src_p = jnp.concatenate(
        [src, jnp.zeros((_NW, pad), jnp.int32)], axis=1
    ).reshape(_NW, _IDX_ROWS, _CHUNK)
    dst_p = jnp.concatenate(
        [dst, jnp.full((_NW, pad), _N, jnp.int32)], axis=1
    ).reshape(_NW, _IDX_ROWS, _CHUNK)
    packed_p = jnp.bitwise_or(src_p, jnp.left_shift(dst_p, 14))

    zeros = jnp.zeros((_ZROWS, _D), _f32)
    ones = jnp.ones((_CHUNK, _D), _f32)

    xs1, hw1 = _mm2(x, W_self1, W_neigh1)
    deg = _make_deg()(dst_p, zeros, ones)[:, :, :16]
    agg1 = _make_sc()(hw1, packed_p, zeros)
    xs2, hw2 = _mid(xs1, agg1, deg, b1.reshape(1, _D), W_self2, W_neigh2)
    agg2 = _make_sc()(hw2, packed_p, zeros)
    return _fin(xs2, agg2, deg, b2.reshape(1, _D))


# speedup vs baseline: 4.1314x; 1.3645x over previous
import functools

import jax
import jax.numpy as jnp
from jax import lax
from jax.experimental import pallas as pl
from jax.experimental.pallas import tpu as pltpu
from jax.experimental.pallas import tpu_sc as plsc

_N = 10000
_E = 320000
_D = 128

_NC = 2
_NS = 16
_NW = _NC * _NS
_CHUNK = 128
_EPT = _E // _NW
_CHUNKS = 79
_IDX_ROWS = 80
_EPT_PAD = _IDX_ROWS * _CHUNK
_ACC_ROWS = 10112
_ZROWS = _ACC_ROWS // _NS

_f32 = jnp.float32


def _sc_body(hw, src, dst, zeros, agg_out, src_v, dst_v, rows_v,
             acc_sh, gsem):
    cid = lax.axis_index("c")
    sid = lax.axis_index("s")
    wid = sid * _NC + cid

    pltpu.sync_copy(zeros, acc_sh.at[pl.ds(sid * _ZROWS, _ZROWS)])
    pltpu.sync_copy(src.at[wid], src_v)
    pltpu.sync_copy(dst.at[wid], dst_v)
    plsc.subcore_barrier()

    def step(j, carry):
        pltpu.async_copy(hw.at[src_v.at[j]], rows_v, gsem).wait()
        pltpu.sync_copy(rows_v, acc_sh.at[dst_v.at[j]], add=True)
        return carry

    lax.fori_loop(0, _CHUNKS, step, 0)
    plsc.subcore_barrier()

    pltpu.sync_copy(acc_sh.at[pl.ds(sid * _ZROWS, _ZROWS)],
                    agg_out.at[cid, pl.ds(sid * _ZROWS, _ZROWS)])


def _make_sc():
    mesh = plsc.VectorSubcoreMesh(core_axis_name="c", subcore_axis_name="s")
    out_type = jax.ShapeDtypeStruct((_NC, _ACC_ROWS, _D), _f32)
    scratch = [
        pltpu.VMEM((_IDX_ROWS, _CHUNK), jnp.int32),
        pltpu.VMEM((_IDX_ROWS, _CHUNK), jnp.int32),
        pltpu.VMEM((_CHUNK, _D), _f32),
        pltpu.VMEM_SHARED((_ACC_ROWS, _D), _f32),
        pltpu.SemaphoreType.DMA,
    ]
    return pl.kernel(_sc_body, out_type=out_type, mesh=mesh,
                     scratch_types=scratch)


def _deg_body(dst, zeros, ones, deg_out, dst_v, ones_v, deg_sh):
    cid = lax.axis_index("c")
    sid = lax.axis_index("s")
    wid = sid * _NC + cid

    pltpu.sync_copy(zeros, deg_sh.at[pl.ds(sid * _ZROWS, _ZROWS)])
    pltpu.sync_copy(dst.at[wid], dst_v)
    pltpu.sync_copy(ones, ones_v)
    plsc.subcore_barrier()

    def step(j, carry):
        pltpu.sync_copy(ones_v, deg_sh.at[dst_v.at[j]], add=True)
        return carry

    lax.fori_loop(0, _CHUNKS, step, 0)
    plsc.subcore_barrier()

    pltpu.sync_copy(deg_sh.at[pl.ds(sid * _ZROWS, _ZROWS)],
                    deg_out.at[cid, pl.ds(sid * _ZROWS, _ZROWS)])


def _make_deg():
    mesh = plsc.VectorSubcoreMesh(core_axis_name="c", subcore_axis_name="s")
    out_type = jax.ShapeDtypeStruct((_NC, _ACC_ROWS, _D), _f32)
    scratch = [
        pltpu.VMEM((_IDX_ROWS, _CHUNK), jnp.int32),
        pltpu.VMEM((_CHUNK, _D), _f32),
        pltpu.VMEM_SHARED((_ACC_ROWS, _D), _f32),
    ]
    return pl.kernel(_deg_body, out_type=out_type, mesh=mesh,
                     scratch_types=scratch)


_BR = 2000


def _mm2_body(x_ref, wa_ref, wb_ref, a_ref, b_ref):
    x = x_ref[...]
    dn = (((1,), (1,)), ((), ()))
    a_ref[...] = lax.dot_general(x, wa_ref[...], dn,
                                 preferred_element_type=_f32)
    b_ref[...] = lax.dot_general(x, wb_ref[...], dn,
                                 preferred_element_type=_f32)


def _mm2(x, wa, wb):
    grid = (_N // _BR,)
    blk_x = pl.BlockSpec((_BR, _D), lambda i: (i, 0))
    blk_w = pl.BlockSpec((_D, _D), lambda i: (0, 0))
    return pl.pallas_call(
        _mm2_body,
        grid=grid,
        in_specs=[blk_x, blk_w, blk_w],
        out_specs=[blk_x, blk_x],
        out_shape=[jax.ShapeDtypeStruct((_N, _D), _f32)] * 2,
    )(x, wa, wb)


def _mid_body(xs_ref, agg_ref, deg_ref, b_ref, wa_ref, wb_ref,
              a_ref, b_out_ref):
    deg = deg_ref[0, :, 0:1] + deg_ref[1, :, 0:1]
    recip = 1.0 / jnp.maximum(deg, 1.0)
    h = xs_ref[...] + (agg_ref[0] + agg_ref[1]) * recip + b_ref[...]
    h = jnp.maximum(h, 0.0)
    dn = (((1,), (1,)), ((), ()))
    a_ref[...] = lax.dot_general(h, wa_ref[...], dn,
                                 preferred_element_type=_f32)
    b_out_ref[...] = lax.dot_general(h, wb_ref[...], dn,
                                     preferred_element_type=_f32)


def _mid(xs, agg, deg, b, wa, wb):
    grid = (_N // _BR,)
    blk_r = pl.BlockSpec((_BR, _D), lambda i: (i, 0))
    blk_a = pl.BlockSpec((_NC, _BR, _D), lambda i: (0, i, 0))
    blk_d = pl.BlockSpec((_NC, _BR, 16), lambda i: (0, i, 0))
    blk_b = pl.BlockSpec((1, _D), lambda i: (0, 0))
    blk_w = pl.BlockSpec((_D, _D), lambda i: (0, 0))
    return pl.pallas_call(
        _mid_body,
        grid=grid,
        in_specs=[blk_r, blk_a, blk_d, blk_b, blk_w, blk_w],
        out_specs=[blk_r, blk_r],
        out_shape=[jax.ShapeDtypeStruct((_N, _D), _f32)] * 2,
    )(xs, agg, deg, b, wa, wb)


def _fin_body(xs_ref, agg_ref, deg_ref, b_ref, o_ref):
    deg = deg_ref[0, :, 0:1] + deg_ref[1, :, 0:1]
    recip = 1.0 / jnp.maximum(deg, 1.0)
    o_ref[...] = xs_ref[...] + (agg_ref[0] + agg_ref[1]) * recip + b_ref[...]


def _fin(xs, agg, deg, b):
    grid = (_N // _BR,)
    blk_r = pl.BlockSpec((_BR, _D), lambda i: (i, 0))
    blk_a = pl.BlockSpec((_NC, _BR, _D), lambda i: (0, i, 0))
    blk_d = pl.BlockSpec((_NC, _BR, 16), lambda i: (0, i, 0))
    blk_b = pl.BlockSpec((1, _D), lambda i: (0, 0))
    return pl.pallas_call(
        _fin_body,
        grid=grid,
        in_specs=[blk_r, blk_a, blk_d, blk_b],
        out_specs=blk_r,
        out_shape=jax.ShapeDtypeStruct((_N, _D), _f32),
    )(xs, agg, deg, b)


def kernel(x, edge_index, W_self1, W_neigh1, b1, W_self2, W_neigh2, b2):
    src = edge_index[0].reshape(_NW, _EPT)
    dst = edge_index[1].reshape(_NW, _EPT)
    pad = _EPT_PAD - _EPT
    src_p = jnp.concatenate(
        [src, jnp.zeros((_NW, pad), jnp.int32)], axis=1
    ).reshape(_NW, _IDX_ROWS, _CHUNK)
    dst_p = jnp.concatenate(
        [dst, jnp.full((_NW, pad), _N, jnp.int32)], axis=1
    ).reshape(_NW, _IDX_ROWS, _CHUNK)

    zeros = jnp.zeros((_ZROWS, _D), _f32)
    ones = jnp.ones((_CHUNK, _D), _f32)

    xs1, hw1 = _mm2(x, W_self1, W_neigh1)
    deg = _make_deg()(dst_p, zeros, ones)[:, :, :16]
    agg1 = _make_sc()(hw1, src_p, dst_p, zeros)
    xs2, hw2 = _mid(xs1, agg1, deg, b1.reshape(1, _D), W_self2, W_neigh2)
    agg2 = _make_sc()(hw2, src_p, dst_p, zeros)
    return _fin(xs2, agg2, deg, b2.reshape(1, _D))


# speedup vs baseline: 4.1323x; 1.0002x over previous
import jax
import jax.numpy as jnp
from jax import lax
from jax.experimental import pallas as pl
from jax.experimental.pallas import tpu as pltpu
from jax.experimental.pallas import tpu_sc as plsc

_N = 10000
_E = 320000
_D = 128

_NC = 2
_NS = 16
_NW = _NC * _NS
_CHUNK = 128
_EPT = _E // _NW
_CHUNKS = 79
_IDX_ROWS = 80
_EPT_PAD = _IDX_ROWS * _CHUNK
_ACC_ROWS = 10112
_ZROWS = _ACC_ROWS // _NS

_f32 = jnp.float32


def _sc_body(hw, src, dst, zeros, agg_out, src_v, dst_v, rows_v,
             acc_sh, gsem):
    cid = lax.axis_index("c")
    sid = lax.axis_index("s")
    wid = sid * _NC + cid

    pltpu.sync_copy(zeros, acc_sh.at[pl.ds(sid * _ZROWS, _ZROWS)])
    pltpu.sync_copy(src.at[wid], src_v)
    pltpu.sync_copy(dst.at[wid], dst_v)
    plsc.subcore_barrier()

    def step(j, carry):
        pltpu.async_copy(hw.at[src_v.at[j]], rows_v, gsem).wait()
        pltpu.sync_copy(rows_v, acc_sh.at[dst_v.at[j]], add=True)
        return carry

    lax.fori_loop(0, _CHUNKS, step, 0)
    plsc.subcore_barrier()

    pltpu.sync_copy(acc_sh.at[pl.ds(sid * _ZROWS, _ZROWS)],
                    agg_out.at[cid, pl.ds(sid * _ZROWS, _ZROWS)])


def _make_sc():
    mesh = plsc.VectorSubcoreMesh(core_axis_name="c", subcore_axis_name="s")
    out_type = jax.ShapeDtypeStruct((_NC, _ACC_ROWS, _D), _f32)
    scratch = [
        pltpu.VMEM((_IDX_ROWS, _CHUNK), jnp.int32),
        pltpu.VMEM((_IDX_ROWS, _CHUNK), jnp.int32),
        pltpu.VMEM((_CHUNK, _D), _f32),
        pltpu.VMEM_SHARED((_ACC_ROWS, _D), _f32),
        pltpu.SemaphoreType.DMA,
    ]
    return pl.kernel(_sc_body, out_type=out_type, mesh=mesh,
                     scratch_types=scratch)


def _deg_body(dst, zeros, ones, deg_out, dst_v, ones_v, deg_sh):
    cid = lax.axis_index("c")
    sid = lax.axis_index("s")
    wid = sid * _NC + cid

    pltpu.sync_copy(zeros, deg_sh.at[pl.ds(sid * _ZROWS, _ZROWS)])
    pltpu.sync_copy(dst.at[wid], dst_v)
    pltpu.sync_copy(ones, ones_v)
    plsc.subcore_barrier()

    def step(j, carry):
        pltpu.sync_copy(ones_v, deg_sh.at[dst_v.at[j]], add=True)
        return carry

    lax.fori_loop(0, _CHUNKS, step, 0)
    plsc.subcore_barrier()

    pltpu.sync_copy(deg_sh.at[pl.ds(sid * _ZROWS, _ZROWS)],
                    deg_out.at[cid, pl.ds(sid * _ZROWS, _ZROWS)])


def _make_deg():
    mesh = plsc.VectorSubcoreMesh(core_axis_name="c", subcore_axis_name="s")
    out_type = jax.ShapeDtypeStruct((_NC, _ACC_ROWS, _D), _f32)
    scratch = [
        pltpu.VMEM((_IDX_ROWS, _CHUNK), jnp.int32),
        pltpu.VMEM((_CHUNK, _D), _f32),
        pltpu.VMEM_SHARED((_ACC_ROWS, _D), _f32),
    ]
    return pl.kernel(_deg_body, out_type=out_type, mesh=mesh,
                     scratch_types=scratch)


_BR = 2000


def _mm2_body(x_ref, wa_ref, wb_ref, a_ref, b_ref):
    x = x_ref[...]
    dn = (((1,), (1,)), ((), ()))
    a_ref[...] = lax.dot_general(x, wa_ref[...], dn,
                                 preferred_element_type=_f32)
    b_ref[...] = lax.dot_general(x, wb_ref[...], dn,
                                 preferred_element_type=_f32)


def _mm2(x, wa, wb):
    grid = (_N // _BR,)
    blk_x = pl.BlockSpec((_BR, _D), lambda i: (i, 0))
    blk_w = pl.BlockSpec((_D, _D), lambda i: (0, 0))
    return pl.pallas_call(
        _mm2_body,
        grid=grid,
        in_specs=[blk_x, blk_w, blk_w],
        out_specs=[blk_x, blk_x],
        out_shape=[jax.ShapeDtypeStruct((_N, _D), _f32)] * 2,
    )(x, wa, wb)


def _mid_body(xs_ref, agg_ref, deg_ref, b_ref, wa_ref, wb_ref,
              a_ref, b_out_ref):
    deg = deg_ref[0, :, 0:1] + deg_ref[1, :, 0:1]
    recip = 1.0 / jnp.maximum(deg, 1.0)
    h = xs_ref[...] + (agg_ref[0] + agg_ref[1]) * recip + b_ref[...]
    h = jnp.maximum(h, 0.0)
    dn = (((1,), (1,)), ((), ()))
    a_ref[...] = lax.dot_general(h, wa_ref[...], dn,
                                 preferred_element_type=_f32)
    b_out_ref[...] = lax.dot_general(h, wb_ref[...], dn,
                                     preferred_element_type=_f32)


def _mid(xs, agg, deg, b, wa, wb):
    grid = (_N // _BR,)
    blk_r = pl.BlockSpec((_BR, _D), lambda i: (i, 0))
    blk_a = pl.BlockSpec((_NC, _BR, _D), lambda i: (0, i, 0))
    blk_d = pl.BlockSpec((_NC, _BR, 16), lambda i: (0, i, 0))
    blk_b = pl.BlockSpec((1, _D), lambda i: (0, 0))
    blk_w = pl.BlockSpec((_D, _D), lambda i: (0, 0))
    return pl.pallas_call(
        _mid_body,
        grid=grid,
        in_specs=[blk_r, blk_a, blk_d, blk_b, blk_w, blk_w],
        out_specs=[blk_r, blk_r],
        out_shape=[jax.ShapeDtypeStruct((_N, _D), _f32)] * 2,
    )(xs, agg, deg, b, wa, wb)


def _fin_body(xs_ref, agg_ref, deg_ref, b_ref, o_ref):
    deg = deg_ref[0, :, 0:1] + deg_ref[1, :, 0:1]
    recip = 1.0 / jnp.maximum(deg, 1.0)
    o_ref[...] = xs_ref[...] + (agg_ref[0] + agg_ref[1]) * recip + b_ref[...]


def _fin(xs, agg, deg, b):
    grid = (_N // _BR,)
    blk_r = pl.BlockSpec((_BR, _D), lambda i: (i, 0))
    blk_a = pl.BlockSpec((_NC, _BR, _D), lambda i: (0, i, 0))
    blk_d = pl.BlockSpec((_NC, _BR, 16), lambda i: (0, i, 0))
    blk_b = pl.BlockSpec((1, _D), lambda i: (0, 0))
    return pl.pallas_call(
        _fin_body,
        grid=grid,
        in_specs=[blk_r, blk_a, blk_d, blk_b],
        out_specs=blk_r,
        out_shape=jax.ShapeDtypeStruct((_N, _D), _f32),
    )(xs, agg, deg, b)


def kernel(x, edge_index, W_self1, W_neigh1, b1, W_self2, W_neigh2, b2):
    src = edge_index[0].reshape(_NW, _EPT)
    dst = edge_index[1].reshape(_NW, _EPT)
    pad = _EPT_PAD - _EPT
    src_p = jnp.concatenate(
        [src, jnp.zeros((_NW, pad), jnp.int32)], axis=1
    ).reshape(_NW, _IDX_ROWS, _CHUNK)
    dst_p = jnp.concatenate(
        [dst, jnp.full((_NW, pad), _N, jnp.int32)], axis=1
    ).reshape(_NW, _IDX_ROWS, _CHUNK)

    zeros = jnp.zeros((_ZROWS, _D), _f32)
    ones = jnp.ones((_CHUNK, _D), _f32)

    xs1, hw1 = _mm2(x, W_self1, W_neigh1)
    deg = _make_deg()(dst_p, zeros, ones)[:, :, :16]
    agg1 = _make_sc()(hw1, src_p, dst_p, zeros)
    xs2, hw2 = _mid(xs1, agg1, deg, b1.reshape(1, _D), W_self2, W_neigh2)
    agg2 = _make_sc()(hw2, src_p, dst_p, zeros)
    return _fin(xs2, agg2, deg, b2.reshape(1, _D))
